# baseline (speedup 1.0000x reference)
import functools

import jax
import jax.numpy as jnp
import numpy as np
from jax.experimental import pallas as pl
from jax.experimental.pallas import tpu as pltpu


def _kimia_body(x_ref, xw_ref, wkvq_ref, wk2_ref,
                w1_ref, w2_ref, temb_ref, sins_ref, o_ref, k_scr, v_scr):
    f32 = jnp.float32
    NI = temb_ref.shape[0] - 1
    D = w1_ref.shape[0]
    DK = D

    bf16 = jnp.bfloat16

    biases = jnp.dot(sins_ref[...], wk2_ref[...], preferred_element_type=f32)

    kv0 = jnp.dot(x_ref[...], xw_ref[...], preferred_element_type=f32)
    k_scr[0] = kv0[:, :DK]
    v0 = kv0[:, DK:]
    v_scr[0] = v0
    A = v0

    for t in range(NI - 1):
        h = jnp.dot(A.astype(bf16), w1_ref[...],
                    preferred_element_type=f32) + temb_ref[t]
        Bt = jnp.dot(jax.nn.gelu(h).astype(bf16), w2_ref[...],
                     preferred_element_type=f32)
        kvq = jnp.dot(Bt.astype(bf16), wkvq_ref[...],
                      preferred_element_type=f32)
        k_scr[t + 1] = kvq[:, :DK] + biases[t]
        v_scr[t + 1] = kvq[:, DK:DK + D]
        q = kvq[:, DK + D:]

        n = t + 2
        svals = [jnp.sum(q * k_scr[j], axis=-1, keepdims=True)
                 for j in range(n)]
        m = svals[0]
        for j in range(1, n):
            m = jnp.maximum(m, svals[j])
        evals = [jnp.exp(s - m) for s in svals]
        den = evals[0]
        for j in range(1, n):
            den = den + evals[j]
        r = 1.0 / den
        A = (evals[0] * r) * v_scr[0]
        for j in range(1, n):
            A = A + (evals[j] * r) * v_scr[j]

    h = jnp.dot(A.astype(bf16), w1_ref[...],
                preferred_element_type=f32) + temb_ref[NI]
    o_ref[...] = jnp.dot(jax.nn.gelu(h).astype(bf16), w2_ref[...],
                         preferred_element_type=f32)


@functools.partial(jax.jit, static_argnames=("interpret",))
def kernel(x, Wik, Wiv, Wq, Wk, Wv, W1, W2, t_emb, interpret=False):
    B, D = x.shape
    DK = Wq.shape[1]
    TR = Wk.shape[0] - D
    NI = t_emb.shape[0] - 1
    dt = x.dtype

    scale = np.float32(1.0 / np.sqrt(DK))
    bf16 = jnp.bfloat16
    wikp = jnp.pad(Wik, ((0, 0), (0, DK - Wik.shape[1])))
    xw = jnp.concatenate([wikp, Wiv], axis=1).astype(bf16)
    wkvq = jnp.concatenate([Wk[:D], Wv, Wq * scale],
                           axis=1).astype(bf16)
    w1b = W1.astype(bf16)
    w2b = W2.astype(bf16)
    xb16 = x.astype(bf16)
    wk2 = Wk[D:]
    t_enc = jnp.pi * (0.5 ** jnp.arange(TR, dtype=dt))
    tvals = jnp.arange(NI, dtype=dt)
    sins = jnp.sin(tvals[:, None] * t_enc[None, :])

    BB = 256
    grid = (B // BB,)
    full = lambda shape: pl.BlockSpec(shape, lambda i: tuple(0 for _ in shape))

    return pl.pallas_call(
        _kimia_body,
        out_shape=jax.ShapeDtypeStruct((B, D), dt),
        grid=grid,
        in_specs=[
            pl.BlockSpec((BB, D), lambda i: (i, 0)),
            full((D, DK + D)),
            full((D, DK + D + DK)),
            full((TR, DK)),
            full((D, D)),
            full((D, D)),
            full((NI + 1, D)),
            full((NI, TR)),
        ],
        out_specs=pl.BlockSpec((BB, D), lambda i: (i, 0)),
        scratch_shapes=[
            pltpu.VMEM((NI, BB, DK), jnp.float32),
            pltpu.VMEM((NI, BB, D), jnp.float32),
        ],
        compiler_params=pltpu.CompilerParams(
            dimension_semantics=("parallel",),
            vmem_limit_bytes=56 * 1024 * 1024,
        ),
        name="ind_kimia_fused",
        interpret=interpret,
    )(xb16, xw, wkvq, wk2, w1b, w2b, t_emb, sins)
